# baseline (speedup 1.0000x reference)
import functools

import jax
import jax.numpy as jnp
from jax import lax
from jax.experimental import pallas as pl
from jax.experimental.pallas import tpu as pltpu
from jax.experimental.pallas import tpu_sc as plsc

N = 10000
E = 320000
D = 128
NC = 2
NS = 16
NW = NC * NS
K = 96
NCHUNK = 105
EPW = NCHUNK * K
EPAD = NW * EPW
NPAD = 10112
SLAB = NPAD // NS
KSUB = K // 16

_mesh = plsc.VectorSubcoreMesh(
    core_axis_name="c", subcore_axis_name="s", num_cores=NC, num_subcores=NS
)
_sc_params = pltpu.CompilerParams(needs_layout_passes=False)


def _worker_id(cid, sid):
    return sid * NC + cid


def _deg_body(dst_hbm, degw_hbm, dst_v, dl_v):
    cid = lax.axis_index("c")
    sid = lax.axis_index("s")
    wid = _worker_id(cid, sid)
    pltpu.sync_copy(dst_hbm.at[wid], dst_v)

    zero16 = jnp.zeros((16,), jnp.float32)
    ones16 = jnp.ones((16,), jnp.float32)

    @pl.loop(0, NPAD // 16)
    def _zero(i):
        dl_v[pl.ds(i * 16, 16)] = zero16

    @pl.loop(0, NCHUNK)
    def _hist(r):
        for j in range(KSUB):
            idx = dst_v[r, pl.ds(j * 16, 16)]
            plsc.addupdate_scatter(dl_v, [idx], ones16)

    pltpu.sync_copy(dl_v, degw_hbm.at[pl.ds(wid * NPAD, NPAD)])


_deg_kernel = functools.partial(
    pl.kernel,
    out_type=jax.ShapeDtypeStruct((NW * NPAD,), jnp.float32),
    mesh=_mesh,
    compiler_params=_sc_params,
    scratch_types=[
        pltpu.VMEM((NCHUNK, K), jnp.int32),
        pltpu.VMEM((NPAD,), jnp.float32),
    ],
)(_deg_body)


def _prop_body(u_hbm, src_hbm, dst_hbm, zeros_hbm, p_hbm,
               src_v, dst_v, rows0, rows1, acc, gsem0, gsem1, ssem0, ssem1):
    cid = lax.axis_index("c")
    sid = lax.axis_index("s")
    wid = _worker_id(cid, sid)

    pltpu.sync_copy(src_hbm.at[pl.ds(wid * EPW, EPW)], src_v)
    pltpu.sync_copy(dst_hbm.at[wid], dst_v)

    base = sid * SLAB
    pltpu.sync_copy(zeros_hbm.at[pl.ds(base, SLAB)], acc.at[pl.ds(base, SLAB)])
    plsc.subcore_barrier()

    def _sidx(c):
        return src_v.at[pl.ds(c * K, K)]

    def _gather(c, buf, sem):
        pltpu.async_copy(u_hbm.at[_sidx(c)], buf, sem)

    def _gwait(c, buf, sem):
        pltpu.make_async_copy(u_hbm.at[_sidx(c)], buf, sem).wait()

    def _scat(c, buf, sem):
        pltpu.async_copy(buf, acc.at[dst_v.at[c]], sem, add=True)

    def _swait(c, buf, sem):
        pltpu.make_async_copy(buf, acc.at[dst_v.at[c]], sem).wait()

    _gather(0, rows0, gsem0)
    _gather(1, rows1, gsem1)

    @pl.loop(0, (NCHUNK - 3) // 2)
    def _edges(o):
        c0 = o * 2
        c1 = c0 + 1
        _gwait(c0, rows0, gsem0)
        _scat(c0, rows0, ssem0)
        _gwait(c1, rows1, gsem1)
        _scat(c1, rows1, ssem1)
        _swait(c0, rows0, ssem0)
        _gather(c0 + 2, rows0, gsem0)
        _swait(c1, rows1, ssem1)
        _gather(c1 + 2, rows1, gsem1)

    ct = NCHUNK - 3
    _gwait(ct, rows0, gsem0)
    _scat(ct, rows0, ssem0)
    _gwait(ct + 1, rows1, gsem1)
    _scat(ct + 1, rows1, ssem1)
    _swait(ct, rows0, ssem0)
    _gather(ct + 2, rows0, gsem0)
    _gwait(ct + 2, rows0, gsem0)
    _scat(ct + 2, rows0, ssem0)
    _swait(ct + 1, rows1, ssem1)
    _swait(ct + 2, rows0, ssem0)

    plsc.subcore_barrier()
    pltpu.sync_copy(acc.at[pl.ds(base, SLAB)],
                    p_hbm.at[cid, pl.ds(base, SLAB)])


_prop_kernel = functools.partial(
    pl.kernel,
    out_type=jax.ShapeDtypeStruct((NC, NPAD, D), jnp.float32),
    mesh=_mesh,
    compiler_params=_sc_params,
    scratch_types=[
        pltpu.VMEM((EPW,), jnp.int32),
        pltpu.VMEM((NCHUNK, K), jnp.int32),
        pltpu.VMEM((K, D), jnp.float32),
        pltpu.VMEM((K, D), jnp.float32),
        pltpu.VMEM_SHARED((NPAD, D), jnp.float32),
        pltpu.SemaphoreType.DMA,
        pltpu.SemaphoreType.DMA,
        pltpu.SemaphoreType.DMA,
        pltpu.SemaphoreType.DMA,
    ],
)(_prop_body)


def _tca_body(degw_ref, dinv_ref):
    deg = jnp.sum(degw_ref[...], axis=0) + 1.0
    dinv_ref[...] = lax.rsqrt(deg)


def _tc1_body(x_ref, w_ref, dinv_ref, u_ref):
    xw = jnp.dot(x_ref[...], w_ref[...],
                 preferred_element_type=jnp.float32,
                 precision=lax.Precision.HIGHEST)
    u_ref[...] = xw * dinv_ref[...]


def _tc2_body(p_ref, u_ref, dinv_ref, b_ref, w_ref, o_ref):
    dinv = dinv_ref[...]
    psum = p_ref[0, :N, :] + p_ref[1, :N, :] + u_ref[...]
    h = psum * dinv + b_ref[...][None, :]
    h = jnp.maximum(h, 0.0)
    hw = jnp.dot(h, w_ref[...],
                 preferred_element_type=jnp.float32,
                 precision=lax.Precision.HIGHEST)
    o_ref[...] = hw * dinv


def _tc3_body(p_ref, u_ref, dinv_ref, b_ref, o_ref):
    psum = p_ref[0, :N, :] + p_ref[1, :N, :] + u_ref[...]
    logits = psum * dinv_ref[...] + b_ref[...][None, :]
    m = jnp.max(logits, axis=1, keepdims=True)
    e = jnp.exp(logits - m)
    o_ref[...] = e / jnp.sum(e, axis=1, keepdims=True)


_tca = pl.pallas_call(
    _tca_body,
    out_shape=jax.ShapeDtypeStruct((NPAD // 128, 128), jnp.float32),
)

_tc1 = pl.pallas_call(
    _tc1_body,
    out_shape=jax.ShapeDtypeStruct((N, D), jnp.float32),
)

_tc2 = pl.pallas_call(
    _tc2_body,
    out_shape=jax.ShapeDtypeStruct((N, D), jnp.float32),
)

_tc3 = pl.pallas_call(
    _tc3_body,
    out_shape=jax.ShapeDtypeStruct((N, D), jnp.float32),
)


@jax.jit
def kernel(x, edge_index, W1, b1, W2, b2):
    npad_e = EPAD - E
    lanes = jnp.arange(npad_e, dtype=jnp.int32)
    src = jnp.concatenate(
        [edge_index[0].astype(jnp.int32), lanes % N])
    dst = jnp.concatenate(
        [edge_index[1].astype(jnp.int32),
         N + lanes % (NPAD - N)]).reshape(NW, NCHUNK, K)

    degw = _deg_kernel(dst)
    dinv_lane = _tca(degw.reshape(NW, NPAD // 128, 128))
    dinv = dinv_lane.reshape(NPAD)[:N].reshape(N, 1)

    zeros = jnp.zeros((NPAD, D), jnp.float32)
    u1 = _tc1(x, W1, dinv)
    p = _prop_kernel(u1, src, dst, zeros)
    u2 = _tc2(p, u1, dinv, b1, W2)
    q = _prop_kernel(u2, src, dst, zeros)
    return _tc3(q, u2, dinv, b2)


# speedup vs baseline: 1.2386x; 1.2386x over previous
import functools

import jax
import jax.numpy as jnp
from jax import lax
from jax.experimental import pallas as pl
from jax.experimental.pallas import tpu as pltpu
from jax.experimental.pallas import tpu_sc as plsc

N = 10000
E = 320000
D = 128
NC = 2
NS = 16
NW = NC * NS
K = 96
NCHUNK = 105
EPW = NCHUNK * K
EPAD = NW * EPW
NPAD = 10112
SLAB = NPAD // NS
KSUB = K // 16

_mesh = plsc.VectorSubcoreMesh(
    core_axis_name="c", subcore_axis_name="s", num_cores=NC, num_subcores=NS
)
_sc_params = pltpu.CompilerParams(needs_layout_passes=False)


def _worker_id(cid, sid):
    return sid * NC + cid


def _deg_body(dst_hbm, degw_hbm, dst_v, dl_v):
    cid = lax.axis_index("c")
    sid = lax.axis_index("s")
    wid = _worker_id(cid, sid)
    pltpu.sync_copy(dst_hbm.at[wid], dst_v)

    zero16 = jnp.zeros((16,), jnp.float32)
    ones16 = jnp.ones((16,), jnp.float32)

    @pl.loop(0, NPAD // 16)
    def _zero(i):
        dl_v[pl.ds(i * 16, 16)] = zero16

    @pl.loop(0, NCHUNK)
    def _hist(r):
        for j in range(KSUB):
            idx = dst_v[r, pl.ds(j * 16, 16)]
            plsc.addupdate_scatter(dl_v, [idx], ones16)

    pltpu.sync_copy(dl_v, degw_hbm.at[pl.ds(wid * NPAD, NPAD)])


_deg_kernel = functools.partial(
    pl.kernel,
    out_type=jax.ShapeDtypeStruct((NW * NPAD,), jnp.float32),
    mesh=_mesh,
    compiler_params=_sc_params,
    scratch_types=[
        pltpu.VMEM((NCHUNK, K), jnp.int32),
        pltpu.VMEM((NPAD,), jnp.float32),
    ],
)(_deg_body)


def _prop_body(u_hbm, src_hbm, dst_hbm, zeros_hbm, p_hbm,
               src_v, dst_v, rows0, rows1, acc, gsem0, gsem1):
    cid = lax.axis_index("c")
    sid = lax.axis_index("s")
    wid = _worker_id(cid, sid)

    pltpu.sync_copy(src_hbm.at[pl.ds(wid * EPW, EPW)], src_v)
    pltpu.sync_copy(dst_hbm.at[wid], dst_v)

    base = sid * SLAB
    pltpu.sync_copy(zeros_hbm.at[pl.ds(base, SLAB)], acc.at[pl.ds(base, SLAB)])
    plsc.subcore_barrier()

    def _sidx(c):
        return src_v.at[pl.ds(c * K, K)]

    pltpu.async_copy(u_hbm.at[_sidx(0)], rows0, gsem0)

    @pl.loop(0, (NCHUNK - 1) // 2)
    def _edges(o):
        c0 = o * 2
        c1 = c0 + 1
        pltpu.async_copy(u_hbm.at[_sidx(c1)], rows1, gsem1)
        pltpu.make_async_copy(u_hbm.at[_sidx(c0)], rows0, gsem0).wait()
        pltpu.sync_copy(rows0, acc.at[dst_v.at[c0]], add=True)
        pltpu.async_copy(u_hbm.at[_sidx(c0 + 2)], rows0, gsem0)
        pltpu.make_async_copy(u_hbm.at[_sidx(c1)], rows1, gsem1).wait()
        pltpu.sync_copy(rows1, acc.at[dst_v.at[c1]], add=True)

    last = NCHUNK - 1
    pltpu.make_async_copy(u_hbm.at[_sidx(last)], rows0, gsem0).wait()
    pltpu.sync_copy(rows0, acc.at[dst_v.at[last]], add=True)

    plsc.subcore_barrier()
    pltpu.sync_copy(acc.at[pl.ds(base, SLAB)],
                    p_hbm.at[cid, pl.ds(base, SLAB)])


_prop_kernel = functools.partial(
    pl.kernel,
    out_type=jax.ShapeDtypeStruct((NC, NPAD, D), jnp.float32),
    mesh=_mesh,
    compiler_params=_sc_params,
    scratch_types=[
        pltpu.VMEM((EPW,), jnp.int32),
        pltpu.VMEM((NCHUNK, K), jnp.int32),
        pltpu.VMEM((K, D), jnp.float32),
        pltpu.VMEM((K, D), jnp.float32),
        pltpu.VMEM_SHARED((NPAD, D), jnp.float32),
        pltpu.SemaphoreType.DMA,
        pltpu.SemaphoreType.DMA,
    ],
)(_prop_body)


def _tca_body(degw_ref, dinv_ref):
    deg = jnp.sum(degw_ref[...], axis=0) + 1.0
    dinv_ref[...] = lax.rsqrt(deg)


def _tc1_body(x_ref, w_ref, dinv_ref, u_ref):
    xw = jnp.dot(x_ref[...], w_ref[...],
                 preferred_element_type=jnp.float32,
                 precision=lax.Precision.HIGHEST)
    u_ref[...] = xw * dinv_ref[...]


def _tc2_body(p_ref, u_ref, dinv_ref, b_ref, w_ref, o_ref):
    dinv = dinv_ref[...]
    psum = p_ref[0, :N, :] + p_ref[1, :N, :] + u_ref[...]
    h = psum * dinv + b_ref[...][None, :]
    h = jnp.maximum(h, 0.0)
    hw = jnp.dot(h, w_ref[...],
                 preferred_element_type=jnp.float32,
                 precision=lax.Precision.HIGHEST)
    o_ref[...] = hw * dinv


def _tc3_body(p_ref, u_ref, dinv_ref, b_ref, o_ref):
    psum = p_ref[0, :N, :] + p_ref[1, :N, :] + u_ref[...]
    logits = psum * dinv_ref[...] + b_ref[...][None, :]
    m = jnp.max(logits, axis=1, keepdims=True)
    e = jnp.exp(logits - m)
    o_ref[...] = e / jnp.sum(e, axis=1, keepdims=True)


_tca = pl.pallas_call(
    _tca_body,
    out_shape=jax.ShapeDtypeStruct((NPAD // 128, 128), jnp.float32),
)

_tc1 = pl.pallas_call(
    _tc1_body,
    out_shape=jax.ShapeDtypeStruct((N, D), jnp.float32),
)

_tc2 = pl.pallas_call(
    _tc2_body,
    out_shape=jax.ShapeDtypeStruct((N, D), jnp.float32),
)

_tc3 = pl.pallas_call(
    _tc3_body,
    out_shape=jax.ShapeDtypeStruct((N, D), jnp.float32),
)


@jax.jit
def kernel(x, edge_index, W1, b1, W2, b2):
    npad_e = EPAD - E
    lanes = jnp.arange(npad_e, dtype=jnp.int32)
    src = jnp.concatenate(
        [edge_index[0].astype(jnp.int32), lanes % N])
    dst = jnp.concatenate(
        [edge_index[1].astype(jnp.int32),
         N + lanes % (NPAD - N)]).reshape(NW, NCHUNK, K)

    degw = _deg_kernel(dst)
    dinv_lane = _tca(degw.reshape(NW, NPAD // 128, 128))
    dinv = dinv_lane.reshape(NPAD)[:N].reshape(N, 1)

    zeros = jnp.zeros((NPAD, D), jnp.float32)
    u1 = _tc1(x, W1, dinv)
    p = _prop_kernel(u1, src, dst, zeros)
    u2 = _tc2(p, u1, dinv, b1, W2)
    q = _prop_kernel(u2, src, dst, zeros)
    return _tc3(q, u2, dinv, b2)


# speedup vs baseline: 1.2635x; 1.0201x over previous
import functools

import jax
import jax.numpy as jnp
from jax import lax
from jax.experimental import pallas as pl
from jax.experimental.pallas import tpu as pltpu
from jax.experimental.pallas import tpu_sc as plsc

N = 10000
E = 320000
D = 128
NC = 2
NS = 16
NW = NC * NS
K = 96
NCHUNK = 105
EPW = NCHUNK * K
EPAD = NW * EPW
NPAD = 10112
SLAB = NPAD // NS
KSUB = K // 16

_mesh = plsc.VectorSubcoreMesh(
    core_axis_name="c", subcore_axis_name="s", num_cores=NC, num_subcores=NS
)
_sc_params = pltpu.CompilerParams(needs_layout_passes=False)


def _worker_id(cid, sid):
    return sid * NC + cid


def _deg_body(dst_hbm, degw_hbm, dst_v, dl_v):
    cid = lax.axis_index("c")
    sid = lax.axis_index("s")
    wid = _worker_id(cid, sid)
    pltpu.sync_copy(dst_hbm.at[wid], dst_v)

    zero16 = jnp.zeros((16,), jnp.float32)
    ones16 = jnp.ones((16,), jnp.float32)

    @pl.loop(0, NPAD // 16)
    def _zero(i):
        dl_v[pl.ds(i * 16, 16)] = zero16

    @pl.loop(0, NCHUNK)
    def _hist(r):
        for j in range(KSUB):
            idx = dst_v[r, pl.ds(j * 16, 16)]
            plsc.addupdate_scatter(dl_v, [idx], ones16)

    pltpu.sync_copy(dl_v, degw_hbm.at[pl.ds(wid * NPAD, NPAD)])


_deg_kernel = functools.partial(
    pl.kernel,
    out_type=jax.ShapeDtypeStruct((NW * NPAD,), jnp.float32),
    mesh=_mesh,
    compiler_params=_sc_params,
    scratch_types=[
        pltpu.VMEM((NCHUNK, K), jnp.int32),
        pltpu.VMEM((NPAD,), jnp.float32),
    ],
)(_deg_body)


def _prop_body(u_hbm, src_hbm, dst_hbm, zeros_hbm, p_hbm,
               src_v, dst_v, rows0, rows1, acc, gsem0, gsem1):
    cid = lax.axis_index("c")
    sid = lax.axis_index("s")
    wid = _worker_id(cid, sid)

    pltpu.sync_copy(src_hbm.at[pl.ds(wid * EPW, EPW)], src_v)
    pltpu.sync_copy(dst_hbm.at[wid], dst_v)

    base = sid * SLAB
    pltpu.sync_copy(zeros_hbm.at[pl.ds(base, SLAB)], acc.at[pl.ds(base, SLAB)])
    plsc.subcore_barrier()

    def _sidx(c):
        return src_v.at[pl.ds(c * K, K)]

    pltpu.async_copy(u_hbm.at[_sidx(0)], rows0, gsem0)

    @pl.loop(0, (NCHUNK - 1) // 2)
    def _edges(o):
        c0 = o * 2
        c1 = c0 + 1
        pltpu.async_copy(u_hbm.at[_sidx(c1)], rows1, gsem1)
        pltpu.make_async_copy(u_hbm.at[_sidx(c0)], rows0, gsem0).wait()
        pltpu.sync_copy(rows0, acc.at[dst_v.at[c0]], add=True)
        pltpu.async_copy(u_hbm.at[_sidx(c0 + 2)], rows0, gsem0)
        pltpu.make_async_copy(u_hbm.at[_sidx(c1)], rows1, gsem1).wait()
        pltpu.sync_copy(rows1, acc.at[dst_v.at[c1]], add=True)

    last = NCHUNK - 1
    pltpu.make_async_copy(u_hbm.at[_sidx(last)], rows0, gsem0).wait()
    pltpu.sync_copy(rows0, acc.at[dst_v.at[last]], add=True)

    plsc.subcore_barrier()
    pltpu.sync_copy(acc.at[pl.ds(base, SLAB)],
                    p_hbm.at[cid, pl.ds(base, SLAB)])


_prop_kernel = functools.partial(
    pl.kernel,
    out_type=jax.ShapeDtypeStruct((NC, NPAD, D), jnp.float32),
    mesh=_mesh,
    compiler_params=_sc_params,
    scratch_types=[
        pltpu.VMEM((EPW,), jnp.int32),
        pltpu.VMEM((NCHUNK, K), jnp.int32),
        pltpu.VMEM((K, D), jnp.float32),
        pltpu.VMEM((K, D), jnp.float32),
        pltpu.VMEM_SHARED((NPAD, D), jnp.float32),
        pltpu.SemaphoreType.DMA,
        pltpu.SemaphoreType.DMA,
    ],
)(_prop_body)


def _tca_body(degw_ref, dinv_ref):
    deg = jnp.sum(degw_ref[...], axis=0) + 1.0
    dinv_ref[...] = lax.rsqrt(deg)


def _tc1_body(x_ref, w_ref, dinv_ref, u_ref):
    xw = jnp.dot(x_ref[...], w_ref[...],
                 preferred_element_type=jnp.float32)
    u_ref[...] = xw * dinv_ref[...]


def _tc2_body(p_ref, u_ref, dinv_ref, b_ref, w_ref, o_ref):
    dinv = dinv_ref[...]
    psum = p_ref[0, :N, :] + p_ref[1, :N, :] + u_ref[...]
    h = psum * dinv + b_ref[...][None, :]
    h = jnp.maximum(h, 0.0)
    hw = jnp.dot(h, w_ref[...],
                 preferred_element_type=jnp.float32)
    o_ref[...] = hw * dinv


def _tc3_body(p_ref, u_ref, dinv_ref, b_ref, o_ref):
    psum = p_ref[0, :N, :] + p_ref[1, :N, :] + u_ref[...]
    logits = psum * dinv_ref[...] + b_ref[...][None, :]
    m = jnp.max(logits, axis=1, keepdims=True)
    e = jnp.exp(logits - m)
    o_ref[...] = e / jnp.sum(e, axis=1, keepdims=True)


_tca = pl.pallas_call(
    _tca_body,
    out_shape=jax.ShapeDtypeStruct((NPAD // 128, 128), jnp.float32),
)

_tc1 = pl.pallas_call(
    _tc1_body,
    out_shape=jax.ShapeDtypeStruct((N, D), jnp.float32),
)

_tc2 = pl.pallas_call(
    _tc2_body,
    out_shape=jax.ShapeDtypeStruct((N, D), jnp.float32),
)

_tc3 = pl.pallas_call(
    _tc3_body,
    out_shape=jax.ShapeDtypeStruct((N, D), jnp.float32),
)


@jax.jit
def kernel(x, edge_index, W1, b1, W2, b2):
    npad_e = EPAD - E
    lanes = jnp.arange(npad_e, dtype=jnp.int32)
    src = jnp.concatenate(
        [edge_index[0].astype(jnp.int32), lanes % N])
    dst = jnp.concatenate(
        [edge_index[1].astype(jnp.int32),
         N + lanes % (NPAD - N)]).reshape(NW, NCHUNK, K)

    degw = _deg_kernel(dst)
    dinv_lane = _tca(degw.reshape(NW, NPAD // 128, 128))
    dinv = dinv_lane.reshape(NPAD)[:N].reshape(N, 1)

    zeros = jnp.zeros((NPAD, D), jnp.float32)
    u1 = _tc1(x, W1, dinv)
    p = _prop_kernel(u1, src, dst, zeros)
    u2 = _tc2(p, u1, dinv, b1, W2)
    q = _prop_kernel(u2, src, dst, zeros)
    return _tc3(q, u2, dinv, b2)
